# Initial kernel scaffold; baseline (speedup 1.0000x reference)
import jax
import jax.numpy as jnp
from jax.experimental import pallas as pl


def kernel(adj_indices, adj_values, embedding):
    raise NotImplementedError("write your pallas kernel here")


# speedup vs baseline: 3.1318x; 3.1318x over previous
import functools

import jax
import jax.numpy as jnp
from jax import lax
from jax.experimental import pallas as pl
from jax.experimental.pallas import tpu as pltpu
from jax.experimental.pallas import tpu_sc as plsc

EMB = 64
B = 128
NC = 2
NS = 16
NW = NC * NS
NCHUNK = 4
TRASH = 128


def _scale_kernel(nnz_pad, n_rows, col, val, x):
    nbatch = nnz_pad // (NW * B)
    mesh = plsc.VectorSubcoreMesh(
        core_axis_name="c", subcore_axis_name="s", num_cores=NC,
        num_subcores=NS)

    @functools.partial(
        pl.kernel,
        out_type=jax.ShapeDtypeStruct((nnz_pad, EMB), jnp.float32),
        mesh=mesh,
        scratch_types=[
            pltpu.VMEM((B,), jnp.int32),
            pltpu.VMEM((B,), jnp.float32),
            pltpu.VMEM((B, EMB), jnp.float32),
            pltpu.SemaphoreType.DMA,
        ],
        compiler_params=pltpu.CompilerParams(use_tc_tiling_on_sc=False),
    )
    def k(col_hbm, val_hbm, x_hbm, out_hbm, idx_v, val_v, rows_v, sem):
        wid = lax.axis_index("s") * NC + lax.axis_index("c")
        base_w = wid * nbatch * B

        def body(i, carry):
            base = base_w + i * B
            pltpu.sync_copy(col_hbm.at[pl.ds(base, B)], idx_v)
            pltpu.sync_copy(val_hbm.at[pl.ds(base, B)], val_v)
            pltpu.async_copy(x_hbm.at[idx_v], rows_v, sem).wait()

            def scale_g(g, c2):
                v16 = val_v[pl.ds(g * 16, 16)]
                for l in range(16):
                    v = v16[l]
                    e = g * 16 + l
                    for j in range(EMB // 16):
                        sl = pl.ds(j * 16, 16)
                        rows_v[e, sl] = rows_v[e, sl] * v
                return c2

            lax.fori_loop(0, B // 16, scale_g, 0)
            pltpu.sync_copy(rows_v, out_hbm.at[pl.ds(base, B)])
            return carry

        lax.fori_loop(0, nbatch, body, 0)

    return k(col, val, x)


def _scatter_kernel(nnz_pad, n_rows, row, scaled):
    chunk_rows = n_rows // NCHUNK
    acc_rows = chunk_rows + TRASH
    nzcopy = acc_rows // B
    zper = nzcopy // NS + 1
    nbatch = nnz_pad // (NS * B)
    dump_rows = chunk_rows // NS
    mesh = plsc.VectorSubcoreMesh(
        core_axis_name="c", subcore_axis_name="s", num_cores=NC,
        num_subcores=NS)

    @functools.partial(
        pl.kernel,
        out_type=jax.ShapeDtypeStruct((n_rows, EMB), jnp.float32),
        mesh=mesh,
        scratch_types=[
            pltpu.VMEM((B,), jnp.int32),
            pltpu.VMEM((B,), jnp.int32),
            pltpu.VMEM((B, EMB), jnp.float32),
            pltpu.VMEM((B, EMB), jnp.float32),
            pltpu.VMEM_SHARED((acc_rows, EMB), jnp.float32),
        ],
        compiler_params=pltpu.CompilerParams(use_tc_tiling_on_sc=False),
    )
    def k(row_hbm, scaled_hbm, y_hbm, idx_v, dst_v, rows_v, zero_v, acc_sh):
        cid = lax.axis_index("c")
        sid = lax.axis_index("s")

        def zinit(e, c2):
            for j in range(EMB // 16):
                zero_v[e, pl.ds(j * 16, 16)] = jnp.zeros((16,), jnp.float32)
            return c2

        lax.fori_loop(0, B, zinit, 0)

        for p in range(NCHUNK // NC):
            chunk = cid + NC * p

            def zbody(j, c2):
                ci = sid * zper + j

                @pl.when(ci < nzcopy)
                def _():
                    pltpu.sync_copy(zero_v, acc_sh.at[pl.ds(ci * B, B)])

                return c2

            lax.fori_loop(0, zper, zbody, 0)
            plsc.subcore_barrier()

            def bbody(i, carry):
                base = (sid * nbatch + i) * B
                pltpu.sync_copy(row_hbm.at[pl.ds(base, B)], idx_v)
                pltpu.sync_copy(scaled_hbm.at[pl.ds(base, B)], rows_v)
                for kk in range(B // 16):
                    sl = pl.ds(kk * 16, 16)
                    r = idx_v[sl]
                    d = r - chunk * chunk_rows
                    inr = (d >= 0) & (d < chunk_rows)
                    trash = (chunk_rows + (kk % (TRASH // 16)) * 16
                             + lax.iota(jnp.int32, 16))
                    dst_v[sl] = jnp.where(inr, d, trash)
                pltpu.sync_copy(rows_v, acc_sh.at[dst_v], add=True)
                return carry

            lax.fori_loop(0, nbatch, bbody, 0)
            plsc.subcore_barrier()

            pltpu.sync_copy(
                acc_sh.at[pl.ds(sid * dump_rows, dump_rows)],
                y_hbm.at[pl.ds(chunk * chunk_rows + sid * dump_rows,
                               dump_rows)])
            plsc.subcore_barrier()

    return k(row, scaled)


def _spmm(nnz_pad, n_rows, row, col, val, x):
    scaled = _scale_kernel(nnz_pad, n_rows, col, val, x)
    return _scatter_kernel(nnz_pad, n_rows, row, scaled)


def _combine_kernel(x0, x1, x2):
    n_rows = x0.shape[0]
    blk = 2048

    def body(a_ref, b_ref, c_ref, o_ref):
        o_ref[...] = (a_ref[...] + b_ref[...] + c_ref[...]) * (1.0 / 3.0)

    return pl.pallas_call(
        body,
        out_shape=jax.ShapeDtypeStruct(x0.shape, x0.dtype),
        grid=(n_rows // blk,),
        in_specs=[pl.BlockSpec((blk, EMB), lambda i: (i, 0))] * 3,
        out_specs=pl.BlockSpec((blk, EMB), lambda i: (i, 0)),
    )(x0, x1, x2)


def kernel(adj_indices, adj_values, embedding):
    nnz = adj_values.shape[0]
    n_rows = embedding.shape[0]
    span = NW * B
    nnz_pad = ((nnz + span - 1) // span) * span
    pad = nnz_pad - nnz
    row = jnp.pad(adj_indices[0].astype(jnp.int32), (0, pad))
    col = jnp.pad(adj_indices[1].astype(jnp.int32), (0, pad))
    val = jnp.pad(adj_values, (0, pad))

    x1 = _spmm(nnz_pad, n_rows, row, col, val, embedding)
    x2 = _spmm(nnz_pad, n_rows, row, col, val, x1)
    return _combine_kernel(embedding, x1, x2)


# speedup vs baseline: 5.8793x; 1.8773x over previous
import functools

import jax
import jax.numpy as jnp
from jax import lax
from jax.experimental import pallas as pl
from jax.experimental.pallas import tpu as pltpu
from jax.experimental.pallas import tpu_sc as plsc

EMB = 64
B = 128
BLK = 512
NC = 2
NS = 16
NW = NC * NS
NCHUNK = 4
TRASH = 128


def _scale_kernel(nnz_pad, n_rows, col, val, x):
    nblk = nnz_pad // (NW * BLK)
    mesh = plsc.VectorSubcoreMesh(
        core_axis_name="c", subcore_axis_name="s", num_cores=NC,
        num_subcores=NS)

    @functools.partial(
        pl.kernel,
        out_type=jax.ShapeDtypeStruct((nnz_pad, EMB), jnp.float32),
        mesh=mesh,
        scratch_types=[
            pltpu.VMEM((NCHUNK, B), jnp.int32),
            pltpu.VMEM((BLK,), jnp.float32),
            pltpu.VMEM((BLK, EMB), jnp.float32),
            pltpu.SemaphoreType.DMA,
            pltpu.SemaphoreType.DMA,
        ],
        compiler_params=pltpu.CompilerParams(use_tc_tiling_on_sc=False),
    )
    def k(col_hbm, val_hbm, x_hbm, out_hbm, idx_v, val_v, rows_v, sem_a,
          sem_g):
        wid = lax.axis_index("s") * NC + lax.axis_index("c")
        base_w = wid * nblk * BLK

        def body(i, carry):
            base = base_w + i * BLK
            dloads = [
                pltpu.async_copy(col_hbm.at[pl.ds(base + q * B, B)],
                                 idx_v.at[q], sem_a)
                for q in range(BLK // B)
            ]
            dloads.append(
                pltpu.async_copy(val_hbm.at[pl.ds(base, BLK)], val_v,
                                 sem_a))
            for d in dloads:
                d.wait()
            descs = [
                pltpu.async_copy(x_hbm.at[idx_v.at[q]],
                                 rows_v.at[pl.ds(q * B, B)], sem_g)
                for q in range(BLK // B)
            ]
            for d in descs:
                d.wait()

            def scale_g(g, c2):
                v16 = val_v[pl.ds(g * 16, 16)]
                for l in range(16):
                    v = v16[l]
                    e = g * 16 + l
                    for j in range(EMB // 16):
                        sl = pl.ds(j * 16, 16)
                        rows_v[e, sl] = rows_v[e, sl] * v
                return c2

            lax.fori_loop(0, BLK // 16, scale_g, 0)
            pltpu.sync_copy(rows_v, out_hbm.at[pl.ds(base, BLK)])
            return carry

        lax.fori_loop(0, nblk, body, 0)

    return k(col, val, x)


def _scatter_kernel(nnz_pad, n_rows, row, scaled):
    chunk_rows = n_rows // NCHUNK
    acc_rows = chunk_rows + TRASH
    nzcopy = acc_rows // B
    zper = nzcopy // NS + 1
    nblk = nnz_pad // (NS * BLK)
    per_tile = nnz_pad // NS
    dump_rows = chunk_rows // NS
    mesh = plsc.VectorSubcoreMesh(
        core_axis_name="c", subcore_axis_name="s", num_cores=NC,
        num_subcores=NS)

    @functools.partial(
        pl.kernel,
        out_type=jax.ShapeDtypeStruct((n_rows, EMB), jnp.float32),
        mesh=mesh,
        scratch_types=[
            pltpu.VMEM((BLK,), jnp.int32),
            pltpu.VMEM((BLK // B, B), jnp.int32),
            pltpu.VMEM((BLK, EMB), jnp.float32),
            pltpu.VMEM((B, EMB), jnp.float32),
            pltpu.VMEM_SHARED((acc_rows, EMB), jnp.float32),
            pltpu.SemaphoreType.DMA,
            pltpu.SemaphoreType.DMA,
        ],
        compiler_params=pltpu.CompilerParams(use_tc_tiling_on_sc=False),
    )
    def k(row_hbm, scaled_hbm, y_hbm, idx_v, dst_v, rows_v, zero_v,
          acc_sh, sem_a, sem_s):
        cid = lax.axis_index("c")
        sid = lax.axis_index("s")

        def zinit(e, c2):
            for j in range(EMB // 16):
                zero_v[e, pl.ds(j * 16, 16)] = jnp.zeros((16,), jnp.float32)
            return c2

        lax.fori_loop(0, B, zinit, 0)

        for p in range(NCHUNK // NC):
            chunk = cid + NC * p

            def zbody(j, c2):
                ci = sid * zper + j

                @pl.when(ci < nzcopy)
                def _():
                    pltpu.sync_copy(zero_v, acc_sh.at[pl.ds(ci * B, B)])

                return c2

            lax.fori_loop(0, zper, zbody, 0)
            plsc.subcore_barrier()

            def bbody(i, carry):
                base = sid * per_tile + i * BLK
                da = pltpu.async_copy(row_hbm.at[pl.ds(base, BLK)],
                                      idx_v, sem_a)
                db = pltpu.async_copy(scaled_hbm.at[pl.ds(base, BLK)],
                                      rows_v, sem_a)
                da.wait()
                db.wait()
                for kk in range(BLK // 16):
                    r = idx_v[pl.ds(kk * 16, 16)]
                    d = r - chunk * chunk_rows
                    inr = (d >= 0) & (d < chunk_rows)
                    trash = (chunk_rows + (kk % (TRASH // 16)) * 16
                             + lax.iota(jnp.int32, 16))
                    dst_v[kk // (B // 16),
                          pl.ds((kk % (B // 16)) * 16, 16)] = (
                              jnp.where(inr, d, trash))
                descs = [
                    pltpu.async_copy(rows_v.at[pl.ds(q * B, B)],
                                     acc_sh.at[dst_v.at[q]], sem_s,
                                     add=True)
                    for q in range(BLK // B)
                ]
                for dd in descs:
                    dd.wait()
                return carry

            lax.fori_loop(0, nblk, bbody, 0)
            plsc.subcore_barrier()

            pltpu.sync_copy(
                acc_sh.at[pl.ds(sid * dump_rows, dump_rows)],
                y_hbm.at[pl.ds(chunk * chunk_rows + sid * dump_rows,
                               dump_rows)])
            plsc.subcore_barrier()

    return k(row, scaled)


def _spmm(nnz_pad, n_rows, row, col, val, x):
    scaled = _scale_kernel(nnz_pad, n_rows, col, val, x)
    return _scatter_kernel(nnz_pad, n_rows, row, scaled)


def _combine_kernel(x0, x1, x2):
    n_rows = x0.shape[0]
    blk = 2048

    def body(a_ref, b_ref, c_ref, o_ref):
        o_ref[...] = (a_ref[...] + b_ref[...] + c_ref[...]) * (1.0 / 3.0)

    return pl.pallas_call(
        body,
        out_shape=jax.ShapeDtypeStruct(x0.shape, x0.dtype),
        grid=(n_rows // blk,),
        in_specs=[pl.BlockSpec((blk, EMB), lambda i: (i, 0))] * 3,
        out_specs=pl.BlockSpec((blk, EMB), lambda i: (i, 0)),
    )(x0, x1, x2)


def kernel(adj_indices, adj_values, embedding):
    nnz = adj_values.shape[0]
    n_rows = embedding.shape[0]
    span = NW * BLK
    nnz_pad = ((nnz + span - 1) // span) * span
    pad = nnz_pad - nnz
    row = jnp.pad(adj_indices[0].astype(jnp.int32), (0, pad))
    col = jnp.pad(adj_indices[1].astype(jnp.int32), (0, pad))
    val = jnp.pad(adj_values, (0, pad))

    x1 = _spmm(nnz_pad, n_rows, row, col, val, embedding)
    x2 = _spmm(nnz_pad, n_rows, row, col, val, x1)
    return _combine_kernel(embedding, x1, x2)


# speedup vs baseline: 5.8809x; 1.0003x over previous
import functools

import jax
import jax.numpy as jnp
from jax import lax
from jax.experimental import pallas as pl
from jax.experimental.pallas import tpu as pltpu
from jax.experimental.pallas import tpu_sc as plsc

EMB = 64
B = 128
BLK = 512
TBLK = 256
NC = 2
NS = 16
NW = NC * NS
NCHUNK = 4
TRASH = 128


def _scale_kernel(nnz_pad, n_rows, col, val, x):
    nblk = nnz_pad // (NW * BLK)
    mesh = plsc.VectorSubcoreMesh(
        core_axis_name="c", subcore_axis_name="s", num_cores=NC,
        num_subcores=NS)

    @functools.partial(
        pl.kernel,
        out_type=jax.ShapeDtypeStruct((nnz_pad, EMB), jnp.float32),
        mesh=mesh,
        scratch_types=[
            pltpu.VMEM((2, NCHUNK, B), jnp.int32),
            pltpu.VMEM((2, BLK), jnp.float32),
            pltpu.VMEM((2, BLK, EMB), jnp.float32),
            pltpu.SemaphoreType.DMA,
            pltpu.SemaphoreType.DMA,
            pltpu.SemaphoreType.DMA,
            pltpu.SemaphoreType.DMA,
            pltpu.SemaphoreType.DMA,
        ],
        compiler_params=pltpu.CompilerParams(use_tc_tiling_on_sc=False),
    )
    def k(col_hbm, val_hbm, x_hbm, out_hbm, idx_v, val_v, rows_v, sem_a,
          sem_b, sem_g0, sem_g1, sem_w):
        wid = lax.axis_index("s") * NC + lax.axis_index("c")
        base_w = wid * nblk * BLK

        def start_loads(base, h, sem):
            descs = [
                pltpu.async_copy(col_hbm.at[pl.ds(base + q * B, B)],
                                 idx_v.at[h, q], sem)
                for q in range(BLK // B)
            ]
            descs.append(
                pltpu.async_copy(val_hbm.at[pl.ds(base, BLK)],
                                 val_v.at[h], sem))
            return descs

        def start_gathers(h, sem):
            return [
                pltpu.async_copy(x_hbm.at[idx_v.at[h, q]],
                                 rows_v.at[h, pl.ds(q * B, B)], sem)
                for q in range(BLK // B)
            ]

        def scale(h):
            def scale_g(g, c2):
                v16 = val_v[h, pl.ds(g * 16, 16)]
                for l in range(16):
                    v = v16[l]
                    e = g * 16 + l
                    for j in range(EMB // 16):
                        sl = pl.ds(j * 16, 16)
                        rows_v[h, e, sl] = rows_v[h, e, sl] * v
                return c2

            lax.fori_loop(0, BLK // 16, scale_g, 0)

        def body(i, carry):
            base0 = base_w + 2 * i * BLK
            base1 = base0 + BLK
            dl0 = start_loads(base0, 0, sem_a)
            dl1 = start_loads(base1, 1, sem_b)
            for d in dl0:
                d.wait()
            dg0 = start_gathers(0, sem_g0)
            for d in dl1:
                d.wait()
            dg1 = start_gathers(1, sem_g1)
            for d in dg0:
                d.wait()
            scale(0)
            dw0 = pltpu.async_copy(rows_v.at[0],
                                   out_hbm.at[pl.ds(base0, BLK)], sem_w)
            for d in dg1:
                d.wait()
            scale(1)
            dw1 = pltpu.async_copy(rows_v.at[1],
                                   out_hbm.at[pl.ds(base1, BLK)], sem_w)
            dw0.wait()
            dw1.wait()
            return carry

        lax.fori_loop(0, nblk // 2, body, 0)

    return k(col, val, x)


def _scatter_kernel(nnz_pad, n_rows, row, scaled):
    chunk_rows = n_rows // NCHUNK
    acc_rows = chunk_rows + TRASH
    nzcopy = acc_rows // B
    zper = nzcopy // NS + 1
    nblk = nnz_pad // (NS * TBLK)
    per_tile = nnz_pad // NS
    dump_rows = chunk_rows // NS
    mesh = plsc.VectorSubcoreMesh(
        core_axis_name="c", subcore_axis_name="s", num_cores=NC,
        num_subcores=NS)

    @functools.partial(
        pl.kernel,
        out_type=jax.ShapeDtypeStruct((n_rows, EMB), jnp.float32),
        mesh=mesh,
        scratch_types=[
            pltpu.VMEM((2, TBLK), jnp.int32),
            pltpu.VMEM((2, TBLK // B, B), jnp.int32),
            pltpu.VMEM((2, TBLK, EMB), jnp.float32),
            pltpu.VMEM((B, EMB), jnp.float32),
            pltpu.VMEM_SHARED((acc_rows, EMB), jnp.float32),
            pltpu.SemaphoreType.DMA,
            pltpu.SemaphoreType.DMA,
            pltpu.SemaphoreType.DMA,
            pltpu.SemaphoreType.DMA,
        ],
        compiler_params=pltpu.CompilerParams(use_tc_tiling_on_sc=False),
    )
    def k(row_hbm, scaled_hbm, y_hbm, idx_v, dst_v, rows_v, zero_v,
          acc_sh, sem_a, sem_b, sem_s0, sem_s1):
        cid = lax.axis_index("c")
        sid = lax.axis_index("s")

        def zinit(e, c2):
            for j in range(EMB // 16):
                zero_v[e, pl.ds(j * 16, 16)] = jnp.zeros((16,), jnp.float32)
            return c2

        lax.fori_loop(0, B, zinit, 0)

        for p in range(NCHUNK // NC):
            chunk = cid + NC * p

            def zbody(j, c2):
                ci = sid * zper + j

                @pl.when(ci < nzcopy)
                def _():
                    pltpu.sync_copy(zero_v, acc_sh.at[pl.ds(ci * B, B)])

                return c2

            lax.fori_loop(0, zper, zbody, 0)
            plsc.subcore_barrier()

            def start_loads(base, h, sem):
                return [
                    pltpu.async_copy(row_hbm.at[pl.ds(base, TBLK)],
                                     idx_v.at[h], sem),
                    pltpu.async_copy(scaled_hbm.at[pl.ds(base, TBLK)],
                                     rows_v.at[h], sem),
                ]

            def compute_dst(h):
                for kk in range(TBLK // 16):
                    r = idx_v[h, pl.ds(kk * 16, 16)]
                    d = r - chunk * chunk_rows
                    inr = (d >= 0) & (d < chunk_rows)
                    trash = (chunk_rows + (kk % (TRASH // 16)) * 16
                             + lax.iota(jnp.int32, 16))
                    dst_v[h, kk // (B // 16),
                          pl.ds((kk % (B // 16)) * 16, 16)] = (
                              jnp.where(inr, d, trash))

            def start_scatters(h, sem):
                return [
                    pltpu.async_copy(rows_v.at[h, pl.ds(q * B, B)],
                                     acc_sh.at[dst_v.at[h, q]], sem,
                                     add=True)
                    for q in range(TBLK // B)
                ]

            def bbody(i, carry):
                base0 = sid * per_tile + 2 * i * TBLK
                base1 = base0 + TBLK
                dl0 = start_loads(base0, 0, sem_a)
                dl1 = start_loads(base1, 1, sem_b)
                for d in dl0:
                    d.wait()
                compute_dst(0)
                ds0 = start_scatters(0, sem_s0)
                for d in dl1:
                    d.wait()
                compute_dst(1)
                ds1 = start_scatters(1, sem_s1)
                for d in ds0:
                    d.wait()
                for d in ds1:
                    d.wait()
                return carry

            lax.fori_loop(0, nblk // 2, bbody, 0)
            plsc.subcore_barrier()

            pltpu.sync_copy(
                acc_sh.at[pl.ds(sid * dump_rows, dump_rows)],
                y_hbm.at[pl.ds(chunk * chunk_rows + sid * dump_rows,
                               dump_rows)])
            plsc.subcore_barrier()

    return k(row, scaled)


def _spmm(nnz_pad, n_rows, row, col, val, x):
    scaled = _scale_kernel(nnz_pad, n_rows, col, val, x)
    return _scatter_kernel(nnz_pad, n_rows, row, scaled)


def _combine_kernel(x0, x1, x2):
    n_rows = x0.shape[0]
    blk = 2048

    def body(a_ref, b_ref, c_ref, o_ref):
        o_ref[...] = (a_ref[...] + b_ref[...] + c_ref[...]) * (1.0 / 3.0)

    return pl.pallas_call(
        body,
        out_shape=jax.ShapeDtypeStruct(x0.shape, x0.dtype),
        grid=(n_rows // blk,),
        in_specs=[pl.BlockSpec((blk, EMB), lambda i: (i, 0))] * 3,
        out_specs=pl.BlockSpec((blk, EMB), lambda i: (i, 0)),
    )(x0, x1, x2)


def kernel(adj_indices, adj_values, embedding):
    nnz = adj_values.shape[0]
    n_rows = embedding.shape[0]
    span = NW * BLK * 2
    nnz_pad = ((nnz + span - 1) // span) * span
    pad = nnz_pad - nnz
    row = jnp.pad(adj_indices[0].astype(jnp.int32), (0, pad))
    col = jnp.pad(adj_indices[1].astype(jnp.int32), (0, pad))
    val = jnp.pad(adj_values, (0, pad))

    x1 = _spmm(nnz_pad, n_rows, row, col, val, embedding)
    x2 = _spmm(nnz_pad, n_rows, row, col, val, x1)
    return _combine_kernel(embedding, x1, x2)
